# Initial kernel scaffold; baseline (speedup 1.0000x reference)
import jax
import jax.numpy as jnp
from jax.experimental import pallas as pl


def kernel(x, edge_index, W1, a_src1, a_dst1, b1, W2, a_src2, a_dst2, b2):
    raise NotImplementedError("write your pallas kernel here")


# speedup vs baseline: 22.8418x; 22.8418x over previous
import functools

import jax
import jax.numpy as jnp
from jax import lax
from jax.experimental import pallas as pl
from jax.experimental.pallas import tpu as pltpu
from jax.experimental.pallas import tpu_sc as plsc

N = 10000
D = 128
E = 320000

NC = 2
NS = 16
NW = NC * NS
EPW = E // NW
CH = 80
NCHUNK = EPW // CH
ROWS_PT = 624
ZR = 104
DEN_PT = 640
DPAD = DEN_PT * NS


def _sc_edge_body(h_hbm, src_hbm, dst_hbm, asrc_hbm, adst_hbm,
                  acc_out, den_out,
                  asrc_v, adst_v, sidx_v, didx_v, w_v, rows_v,
                  zrow_v, zden_v, acc_sh, den_sh, sem):
    c = lax.axis_index("c")
    s = lax.axis_index("s")
    wid = s * NC + c

    pltpu.sync_copy(asrc_hbm, asrc_v)
    pltpu.sync_copy(adst_hbm, adst_v)

    def _zrow(i, carry):
        for r in range(D // 16):
            zrow_v[i, pl.ds(r * 16, 16)] = jnp.zeros((16,), jnp.float32)
        return carry
    lax.fori_loop(0, ZR, _zrow, 0)

    def _zden(i, carry):
        zden_v[pl.ds(i * 16, 16)] = jnp.zeros((16,), jnp.float32)
        return carry
    lax.fori_loop(0, DEN_PT // 16, _zden, 0)

    for j in range(ROWS_PT // ZR):
        pltpu.sync_copy(zrow_v, acc_sh.at[pl.ds(s * ROWS_PT + j * ZR, ZR)])

    @pl.when(s == NS - 1)
    def _zero_tail():
        pltpu.sync_copy(zrow_v.at[pl.ds(0, 16)], acc_sh.at[pl.ds(N - 16, 16)])

    pltpu.sync_copy(zden_v, den_sh.at[pl.ds(s * DEN_PT, DEN_PT)])
    plsc.subcore_barrier()

    def _chunk(t, carry):
        base = wid * EPW + t * CH
        pltpu.sync_copy(src_hbm.at[pl.ds(base, CH)], sidx_v)
        pltpu.sync_copy(dst_hbm.at[pl.ds(base, CH)], didx_v)
        gat = pltpu.async_copy(h_hbm.at[sidx_v], rows_v, sem)

        for k in range(CH // 16):
            sl = pl.ds(k * 16, 16)
            e = (plsc.load_gather(asrc_v, [sidx_v[sl]])
                 + plsc.load_gather(adst_v, [didx_v[sl]]))
            e = jnp.where(e >= 0.0, e, 0.2 * e)
            w_v[sl] = jnp.exp(e)

        pltpu.sync_copy(w_v, den_sh.at[didx_v], add=True)

        gat.wait()

        def _scale(i, carry2):
            wspl = plsc.load_gather(w_v, [jnp.full((16,), i, jnp.int32)])
            for r in range(D // 16):
                rsl = pl.ds(r * 16, 16)
                rows_v[i, rsl] = rows_v[i, rsl] * wspl
            return carry2
        lax.fori_loop(0, CH, _scale, 0)

        pltpu.sync_copy(rows_v, acc_sh.at[didx_v], add=True)
        return carry

    lax.fori_loop(0, NCHUNK, _chunk, 0)
    plsc.subcore_barrier()

    off = s * ROWS_PT
    pltpu.sync_copy(acc_sh.at[pl.ds(off, ROWS_PT)],
                    acc_out.at[c, pl.ds(off, ROWS_PT)])

    @pl.when(s == NS - 1)
    def _wb_tail():
        pltpu.sync_copy(acc_sh.at[pl.ds(N - 16, 16)],
                        acc_out.at[c, pl.ds(N - 16, 16)])

    pltpu.sync_copy(den_sh.at[pl.ds(s * DEN_PT, DEN_PT)],
                    den_out.at[c, pl.ds(s * DEN_PT, DEN_PT)])


@functools.cache
def _get_sc_edge():
  return pl.kernel(
    _sc_edge_body,
    out_type=(jax.ShapeDtypeStruct((NC, N, D), jnp.float32),
              jax.ShapeDtypeStruct((NC, DPAD), jnp.float32)),
    mesh=plsc.VectorSubcoreMesh(core_axis_name="c", subcore_axis_name="s",
                                num_cores=NC, num_subcores=NS),
    compiler_params=pltpu.CompilerParams(needs_layout_passes=False),
    scratch_types=[
        pltpu.VMEM((N,), jnp.float32),
        pltpu.VMEM((N,), jnp.float32),
        pltpu.VMEM((CH,), jnp.int32),
        pltpu.VMEM((CH,), jnp.int32),
        pltpu.VMEM((CH,), jnp.float32),
        pltpu.VMEM((CH, D), jnp.float32),
        pltpu.VMEM((ZR, D), jnp.float32),
        pltpu.VMEM((DEN_PT,), jnp.float32),
        pltpu.VMEM_SHARED((N, D), jnp.float32),
        pltpu.VMEM_SHARED((DPAD,), jnp.float32),
        pltpu.SemaphoreType.DMA,
    ],
  )


def _tc_dense_body(x_ref, w_ref, as_ref, ad_ref, h_ref, asrc_ref, adst_ref, sw_ref):
    h = jnp.dot(x_ref[...], w_ref[...], preferred_element_type=jnp.float32)
    h_ref[...] = h
    asrc = jnp.sum(h * as_ref[...], axis=1, keepdims=True)
    adst = jnp.sum(h * ad_ref[...], axis=1, keepdims=True)
    asrc_ref[...] = asrc
    adst_ref[...] = adst
    e = asrc + adst
    e = jnp.where(e >= 0.0, e, 0.2 * e)
    sw_ref[...] = jnp.exp(e)


@functools.cache
def _get_tc_dense():
  return pl.pallas_call(
    _tc_dense_body,
    out_shape=(jax.ShapeDtypeStruct((N, D), jnp.float32),
               jax.ShapeDtypeStruct((N, 1), jnp.float32),
               jax.ShapeDtypeStruct((N, 1), jnp.float32),
               jax.ShapeDtypeStruct((N, 1), jnp.float32)),
  )


def _tc_combine_body(slope, acc0_ref, acc1_ref, den0_ref, den1_ref,
                     sw_ref, h_ref, b_ref, out_ref):
    den = den0_ref[...] + den1_ref[...] + sw_ref[...] + 1e-16
    num = acc0_ref[...] + acc1_ref[...] + sw_ref[...] * h_ref[...]
    out = num / den + b_ref[...]
    if slope is not None:
        out = jnp.where(out >= 0.0, out, slope * out)
    out_ref[...] = out


@functools.cache
def _make_combine(slope):
    return pl.pallas_call(
        functools.partial(_tc_combine_body, slope),
        out_shape=jax.ShapeDtypeStruct((N, D), jnp.float32),
    )


def _gat_layer(x, src, dst, W, a_s, a_d, b, slope):
    h, asrc, adst, sw = _get_tc_dense()(x, W, a_s.reshape(1, D), a_d.reshape(1, D))
    acc, den = _get_sc_edge()(h, src, dst, asrc.reshape(N), adst.reshape(N))
    return _make_combine(slope)(acc[0], acc[1], den[0, :N].reshape(N, 1),
                                den[1, :N].reshape(N, 1), sw, h, b.reshape(1, D))


def kernel(x, edge_index, W1, a_src1, a_dst1, b1, W2, a_src2, a_dst2, b2):
    src = edge_index[0]
    dst = edge_index[1]
    h1 = _gat_layer(x, src, dst, W1, a_src1, a_dst1, b1, 0.01)
    out = _gat_layer(h1, src, dst, W2, a_src2, a_dst2, b2, None)
    return out


# speedup vs baseline: 33.9541x; 1.4865x over previous
import functools

import jax
import jax.numpy as jnp
from jax import lax
from jax.experimental import pallas as pl
from jax.experimental.pallas import tpu as pltpu
from jax.experimental.pallas import tpu_sc as plsc

N = 10000
D = 128
E = 320000

NC = 2
NS = 16
NW = NC * NS
CH = 128
NCHUNK = E // CH
CPW = 80
ROWS_PT = 624
DEN_PT = 640
DPAD = DEN_PT * NS


def _sc_edge_body(h_hbm, src_hbm, dst_hbm, asrc_hbm, adst_hbm,
                  acc_out, den_out,
                  sidx0, sidx1, didx0, didx1, w0, w1, asg0, asg1, adg0, adg1,
                  rows0, rows1, acc_sh, den_sh,
                  sem_g0, sem_g1, sem_i0, sem_i1, sem_d0, sem_d1,
                  sem_s0, sem_s1):
    c = lax.axis_index("c")
    s = lax.axis_index("s")
    wid = s * NC + c
    sidx = (sidx0, sidx1)
    didx = (didx0, didx1)
    wb = (w0, w1)
    asg = (asg0, asg1)
    adg = (adg0, adg1)
    rows = (rows0, rows1)
    sem_g = (sem_g0, sem_g1)
    sem_i = (sem_i0, sem_i1)
    sem_d = (sem_d0, sem_d1)
    sem_s = (sem_s0, sem_s1)

    def _zrow(i, carry):
        for r in range(D // 16):
            rows0[i, pl.ds(r * 16, 16)] = jnp.zeros((16,), jnp.float32)
        return carry
    lax.fori_loop(0, CH, _zrow, 0)
    for r in range(D // 16):
        w0[pl.ds(r * 16, 16)] = jnp.zeros((16,), jnp.float32)

    for j in range(ROWS_PT // CH):
        pltpu.sync_copy(rows0, acc_sh.at[pl.ds(s * ROWS_PT + j * CH, CH)])
    pltpu.sync_copy(rows0.at[pl.ds(0, ROWS_PT % CH)],
                    acc_sh.at[pl.ds(s * ROWS_PT + (ROWS_PT // CH) * CH,
                                    ROWS_PT % CH)])

    @pl.when(s == NS - 1)
    def _zero_tail():
        pltpu.sync_copy(rows0.at[pl.ds(0, 16)], acc_sh.at[pl.ds(N - 16, 16)])

    for j in range(DEN_PT // CH):
        pltpu.sync_copy(w0, den_sh.at[pl.ds(s * DEN_PT + j * CH, CH)])
    plsc.subcore_barrier()

    def _chunk_base(t):
        gc = wid + NW * t
        base = jnp.where(gc < NCHUNK, gc * CH, 0)
        return gc, pl.multiple_of(base, CH)

    def _issue_idx(t, p):
        _, base = _chunk_base(t)
        pltpu.async_copy(src_hbm.at[pl.ds(base, CH)], sidx[p], sem_i[p])
        pltpu.async_copy(dst_hbm.at[pl.ds(base, CH)], didx[p], sem_i[p])

    def _drain_idx(p):
        d = pltpu.make_async_copy(src_hbm.at[pl.ds(0, CH)], sidx[p], sem_i[p])
        d.wait()
        d.wait()

    def _issue_gather(p):
        pltpu.async_copy(h_hbm.at[sidx[p]], rows[p], sem_g[p])
        pltpu.async_copy(asrc_hbm.at[sidx[p]], asg[p], sem_g[p])
        pltpu.async_copy(adst_hbm.at[didx[p]], adg[p], sem_g[p])

    def _drain_gather(p):
        pltpu.make_async_copy(h_hbm.at[pl.ds(0, CH)], rows[p], sem_g[p]).wait()
        d = pltpu.make_async_copy(asrc_hbm.at[pl.ds(0, CH)], asg[p], sem_g[p])
        d.wait()
        d.wait()

    def _drain_scatters(p):
        pltpu.make_async_copy(h_hbm.at[pl.ds(0, CH)], rows[p], sem_s[p]).wait()
        pltpu.make_async_copy(asrc_hbm.at[pl.ds(0, CH)], wb[p], sem_d[p]).wait()

    def _sub_body(t, p):
        np_ = 1 - p
        gc, _ = _chunk_base(t)
        _drain_gather(p)

        if p == 1:
            _drain_scatters(np_)
        else:
            @pl.when(t >= 1)
            def _():
                _drain_scatters(np_)

        if p == 0:
            _issue_idx(t + 1, np_)
        else:
            @pl.when(t + 1 < CPW)
            def _():
                _issue_idx(t + 1, np_)

        wmask = jnp.where(gc < NCHUNK, 1.0, 0.0).astype(jnp.float32)
        for k in range(CH // 16):
            sl = pl.ds(k * 16, 16)
            e = asg[p][sl] + adg[p][sl]
            e = jnp.where(e >= 0.0, e, 0.2 * e)
            wb[p][sl] = jnp.exp(e) * wmask

        pltpu.async_copy(wb[p], den_sh.at[didx[p]], sem_d[p], add=True)

        def _scale(i, carry2):
            wspl = plsc.load_gather(wb[p], [jnp.full((16,), i, jnp.int32)])
            for r in range(D // 16):
                rsl = pl.ds(r * 16, 16)
                rows[p][i, rsl] = rows[p][i, rsl] * wspl
            return carry2
        lax.fori_loop(0, CH, _scale, 0)

        if p == 0:
            _drain_idx(np_)
            _issue_gather(np_)
        else:
            @pl.when(t + 1 < CPW)
            def _():
                _drain_idx(np_)
                _issue_gather(np_)

        pltpu.async_copy(rows[p], acc_sh.at[didx[p]], sem_s[p], add=True)

    pltpu.sync_copy(src_hbm.at[pl.ds(wid * CH, CH)], sidx[0])
    pltpu.sync_copy(dst_hbm.at[pl.ds(wid * CH, CH)], didx[0])
    _issue_gather(0)

    def _pair(t2, carry):
        _sub_body(2 * t2, 0)
        _sub_body(2 * t2 + 1, 1)
        return carry

    lax.fori_loop(0, CPW // 2, _pair, 0)
    _drain_scatters(1)
    plsc.subcore_barrier()

    off = s * ROWS_PT
    pltpu.sync_copy(acc_sh.at[pl.ds(off, ROWS_PT)],
                    acc_out.at[c, pl.ds(off, ROWS_PT)])

    @pl.when(s == NS - 1)
    def _wb_tail():
        pltpu.sync_copy(acc_sh.at[pl.ds(N - 16, 16)],
                        acc_out.at[c, pl.ds(N - 16, 16)])

    pltpu.sync_copy(den_sh.at[pl.ds(s * DEN_PT, DEN_PT)],
                    den_out.at[c, pl.ds(s * DEN_PT, DEN_PT)])


@functools.cache
def _get_sc_edge():
  return pl.kernel(
    _sc_edge_body,
    out_type=(jax.ShapeDtypeStruct((NC, N, D), jnp.float32),
              jax.ShapeDtypeStruct((NC, DPAD), jnp.float32)),
    mesh=plsc.VectorSubcoreMesh(core_axis_name="c", subcore_axis_name="s",
                                num_cores=NC, num_subcores=NS),
    compiler_params=pltpu.CompilerParams(needs_layout_passes=False),
    scratch_types=[
        pltpu.VMEM((CH,), jnp.int32),
        pltpu.VMEM((CH,), jnp.int32),
        pltpu.VMEM((CH,), jnp.int32),
        pltpu.VMEM((CH,), jnp.int32),
        pltpu.VMEM((CH,), jnp.float32),
        pltpu.VMEM((CH,), jnp.float32),
        pltpu.VMEM((CH,), jnp.float32),
        pltpu.VMEM((CH,), jnp.float32),
        pltpu.VMEM((CH,), jnp.float32),
        pltpu.VMEM((CH,), jnp.float32),
        pltpu.VMEM((CH, D), jnp.float32),
        pltpu.VMEM((CH, D), jnp.float32),
        pltpu.VMEM_SHARED((N, D), jnp.float32),
        pltpu.VMEM_SHARED((DPAD,), jnp.float32),
    ] + [pltpu.SemaphoreType.DMA] * 8,
  )


def _tc_dense_body(x_ref, w_ref, as_ref, ad_ref, h_ref, asrc_ref, adst_ref, sw_ref):
    h = jnp.dot(x_ref[...], w_ref[...], preferred_element_type=jnp.float32)
    h_ref[...] = h
    asrc = jnp.sum(h * as_ref[...], axis=1, keepdims=True)
    adst = jnp.sum(h * ad_ref[...], axis=1, keepdims=True)
    asrc_ref[...] = asrc
    adst_ref[...] = adst
    e = asrc + adst
    e = jnp.where(e >= 0.0, e, 0.2 * e)
    sw_ref[...] = jnp.exp(e)


@functools.cache
def _get_tc_dense():
  return pl.pallas_call(
    _tc_dense_body,
    out_shape=(jax.ShapeDtypeStruct((N, D), jnp.float32),
               jax.ShapeDtypeStruct((N, 1), jnp.float32),
               jax.ShapeDtypeStruct((N, 1), jnp.float32),
               jax.ShapeDtypeStruct((N, 1), jnp.float32)),
  )


def _tc_combine_body(slope, acc0_ref, acc1_ref, den0_ref, den1_ref,
                     sw_ref, h_ref, b_ref, out_ref):
    den = den0_ref[...] + den1_ref[...] + sw_ref[...] + 1e-16
    num = acc0_ref[...] + acc1_ref[...] + sw_ref[...] * h_ref[...]
    out = num / den + b_ref[...]
    if slope is not None:
        out = jnp.where(out >= 0.0, out, slope * out)
    out_ref[...] = out


@functools.cache
def _make_combine(slope):
    return pl.pallas_call(
        functools.partial(_tc_combine_body, slope),
        out_shape=jax.ShapeDtypeStruct((N, D), jnp.float32),
    )


def _gat_layer(x, src, dst, W, a_s, a_d, b, slope):
    h, asrc, adst, sw = _get_tc_dense()(x, W, a_s.reshape(1, D), a_d.reshape(1, D))
    acc, den = _get_sc_edge()(h, src, dst, asrc.reshape(N), adst.reshape(N))
    return _make_combine(slope)(acc[0], acc[1], den[0, :N].reshape(N, 1),
                                den[1, :N].reshape(N, 1), sw, h, b.reshape(1, D))


def kernel(x, edge_index, W1, a_src1, a_dst1, b1, W2, a_src2, a_dst2, b2):
    src = edge_index[0]
    dst = edge_index[1]
    h1 = _gat_layer(x, src, dst, W1, a_src1, a_dst1, b1, 0.01)
    out = _gat_layer(h1, src, dst, W2, a_src2, a_dst2, b2, None)
    return out


# speedup vs baseline: 43.3429x; 1.2765x over previous
import functools

import jax
import jax.numpy as jnp
from jax import lax
from jax.experimental import pallas as pl
from jax.experimental.pallas import tpu as pltpu
from jax.experimental.pallas import tpu_sc as plsc

N = 10000
D = 128
E = 320000

NC = 2
NS = 16
NW = NC * NS
CH = 128
NCHUNK = E // CH
CPW = 80
NB = 8
NBT = CPW // NB
EPAD = NW * CPW * CH
ROWS_PT = 624
DEN_PT = 640
DPAD = DEN_PT * NS


def _sc_edge_body(h_hbm, src_hbm, dst_hbm, asrc_hbm, adst_hbm,
                  acc_out, den_out,
                  sidxB0, sidxB1, didxB0, didxB1, asgB0, asgB1, adgB0, adgB1,
                  wB0, wB1, rows0, rows1, acc_sh, den_sh,
                  sem_g0, sem_g1, sem_i0, sem_i1, sem_a0, sem_a1,
                  sem_d0, sem_d1, sem_s0, sem_s1):
    c = lax.axis_index("c")
    s = lax.axis_index("s")
    wid = s * NC + c
    sidxB = (sidxB0, sidxB1)
    didxB = (didxB0, didxB1)
    asgB = (asgB0, asgB1)
    adgB = (adgB0, adgB1)
    wB = (wB0, wB1)
    rows = (rows0, rows1)
    sem_g = (sem_g0, sem_g1)
    sem_i = (sem_i0, sem_i1)
    sem_a = (sem_a0, sem_a1)
    sem_d = (sem_d0, sem_d1)
    sem_s = (sem_s0, sem_s1)

    def _zrow(i, carry):
        for r in range(D // 16):
            rows0[i, pl.ds(r * 16, 16)] = jnp.zeros((16,), jnp.float32)
        return carry
    lax.fori_loop(0, CH, _zrow, 0)

    for j in range(ROWS_PT // CH):
        pltpu.sync_copy(rows0, acc_sh.at[pl.ds(s * ROWS_PT + j * CH, CH)])
    pltpu.sync_copy(rows0.at[pl.ds(0, ROWS_PT % CH)],
                    acc_sh.at[pl.ds(s * ROWS_PT + (ROWS_PT // CH) * CH,
                                    ROWS_PT % CH)])

    @pl.when(s == NS - 1)
    def _zero_tail():
        pltpu.sync_copy(rows0.at[pl.ds(0, 16)], acc_sh.at[pl.ds(N - 16, 16)])

    for j in range(DEN_PT // CH):
        pltpu.sync_copy(rows0.at[0], den_sh.at[pl.ds(s * DEN_PT + j * CH, CH)])
    plsc.subcore_barrier()


    def _batch_off(b):
        return pl.multiple_of(wid * CPW + NB * b, NB)

    def _issue_idx(b, P):
        off = _batch_off(b)
        pltpu.async_copy(src_hbm.at[pl.ds(off, NB)], sidxB[P], sem_i[P])
        pltpu.async_copy(dst_hbm.at[pl.ds(off, NB)], didxB[P], sem_i[P])

    def _drain_idx(P):
        d = pltpu.make_async_copy(src_hbm.at[pl.ds(0, NB)], sidxB[P], sem_i[P])
        d.wait()
        d.wait()

    def _issue_scores(P):
        for j in range(NB):
            pltpu.async_copy(asrc_hbm.at[sidxB[P].at[j]], asgB[P].at[j],
                             sem_a[P])
            pltpu.async_copy(adst_hbm.at[didxB[P].at[j]], adgB[P].at[j],
                             sem_a[P])

    def _drain_scores(P):
        d = pltpu.make_async_copy(asrc_hbm.at[pl.ds(0, CH)], asgB[P].at[0],
                                  sem_a[P])
        for _ in range(2 * NB):
            d.wait()

    def _issue_rowgather(P, j, q):
        pltpu.async_copy(h_hbm.at[sidxB[P].at[j]], rows[q], sem_g[q])

    def _drain_rowgather(q):
        pltpu.make_async_copy(h_hbm.at[pl.ds(0, CH)], rows[q], sem_g[q]).wait()

    def _issue_rowscatter(P, j, q):
        pltpu.async_copy(rows[q], acc_sh.at[didxB[P].at[j]], sem_s[q],
                         add=True)

    def _drain_rowscatter(q):
        pltpu.make_async_copy(h_hbm.at[pl.ds(0, CH)], rows[q], sem_s[q]).wait()

    def _issue_den(P):
        for j in range(NB):
            pltpu.async_copy(wB[P].at[j], den_sh.at[didxB[P].at[j]],
                             sem_d[P], add=True)

    def _drain_den(P):
        d = pltpu.make_async_copy(asrc_hbm.at[pl.ds(0, CH)], wB[P].at[0],
                                  sem_d[P])
        for _ in range(NB):
            d.wait()

    def _batch_body(b, P, drain_prev, has_next):
        NPp = 1 - P

        def _top_drains():
            _drain_rowscatter(1)
            _drain_den(NPp)
        if drain_prev is None:
            _top_drains()
        else:
            pl.when(drain_prev)(_top_drains)

        _drain_scores(P)

        def _next_idx():
            _issue_idx(b + 1, NPp)
        if has_next is None:
            _next_idx()
        else:
            pl.when(has_next)(_next_idx)

        for j in range(NB):
            q = j & 1
            _drain_rowgather(q)

            gc = wid * CPW + NB * b + j
            wmask = jnp.where(gc < NCHUNK, 1.0, 0.0).astype(jnp.float32)
            for k in range(CH // 16):
                sl = pl.ds(k * 16, 16)
                e = asgB[P][j, sl] + adgB[P][j, sl]
                e = jnp.where(e >= 0.0, e, 0.2 * e)
                wB[P][j, sl] = jnp.exp(e) * wmask

            if j < NB - 1:
                q2 = (j + 1) & 1
                if j > 0:
                    _drain_rowscatter(q2)
                _issue_rowgather(P, j + 1, q2)
            else:
                def _next_batch_head():
                    _drain_idx(NPp)
                    _issue_scores(NPp)
                    _drain_rowscatter(0)
                    _issue_rowgather(NPp, 0, 0)
                if has_next is None:
                    _next_batch_head()
                else:
                    pl.when(has_next)(_next_batch_head)

            def _scale(i4, carry2):
                for u in range(4):
                    i = i4 * 4 + u
                    wspl = plsc.load_gather(
                        wB[P].at[j], [jnp.full((16,), i, jnp.int32)])
                    for r in range(D // 16):
                        rsl = pl.ds(r * 16, 16)
                        rows[q][i, rsl] = rows[q][i, rsl] * wspl
                return carry2
            lax.fori_loop(0, CH // 4, _scale, 0)

            _issue_rowscatter(P, j, q)

        _issue_den(P)

    off0 = _batch_off(0)
    pltpu.sync_copy(src_hbm.at[pl.ds(off0, NB)], sidxB[0])
    pltpu.sync_copy(dst_hbm.at[pl.ds(off0, NB)], didxB[0])
    _issue_scores(0)
    _issue_rowgather(0, 0, 0)

    def _bpair(i, carry):
        _batch_body(2 * i, 0, drain_prev=(i >= 1), has_next=None)
        _batch_body(2 * i + 1, 1, drain_prev=None, has_next=(i < NBT // 2 - 1))
        return carry

    lax.fori_loop(0, NBT // 2, _bpair, 0)
    _drain_rowscatter(0)
    _drain_rowscatter(1)
    _drain_den(1)
    plsc.subcore_barrier()

    off = s * ROWS_PT
    pltpu.sync_copy(acc_sh.at[pl.ds(off, ROWS_PT)],
                    acc_out.at[c, pl.ds(off, ROWS_PT)])

    @pl.when(s == NS - 1)
    def _wb_tail():
        pltpu.sync_copy(acc_sh.at[pl.ds(N - 16, 16)],
                        acc_out.at[c, pl.ds(N - 16, 16)])

    pltpu.sync_copy(den_sh.at[pl.ds(s * DEN_PT, DEN_PT)],
                    den_out.at[c, pl.ds(s * DEN_PT, DEN_PT)])


@functools.cache
def _get_sc_edge():
  return pl.kernel(
    _sc_edge_body,
    out_type=(jax.ShapeDtypeStruct((NC, N, D), jnp.float32),
              jax.ShapeDtypeStruct((NC, DPAD), jnp.float32)),
    mesh=plsc.VectorSubcoreMesh(core_axis_name="c", subcore_axis_name="s",
                                num_cores=NC, num_subcores=NS),
    compiler_params=pltpu.CompilerParams(needs_layout_passes=False),
    scratch_types=[
        pltpu.VMEM((NB, CH), jnp.int32),
        pltpu.VMEM((NB, CH), jnp.int32),
        pltpu.VMEM((NB, CH), jnp.int32),
        pltpu.VMEM((NB, CH), jnp.int32),
        pltpu.VMEM((NB, CH), jnp.float32),
        pltpu.VMEM((NB, CH), jnp.float32),
        pltpu.VMEM((NB, CH), jnp.float32),
        pltpu.VMEM((NB, CH), jnp.float32),
        pltpu.VMEM((NB, CH), jnp.float32),
        pltpu.VMEM((NB, CH), jnp.float32),
        pltpu.VMEM((CH, D), jnp.float32),
        pltpu.VMEM((CH, D), jnp.float32),
        pltpu.VMEM_SHARED((N, D), jnp.float32),
        pltpu.VMEM_SHARED((DPAD,), jnp.float32),
    ] + [pltpu.SemaphoreType.DMA] * 10,
  )


def _tc_dense_body(x_ref, w_ref, as_ref, ad_ref, h_ref, asrc_ref, adst_ref, sw_ref):
    h = jnp.dot(x_ref[...], w_ref[...], preferred_element_type=jnp.float32)
    h_ref[...] = h
    asrc = jnp.sum(h * as_ref[...], axis=1, keepdims=True)
    adst = jnp.sum(h * ad_ref[...], axis=1, keepdims=True)
    asrc_ref[...] = asrc
    adst_ref[...] = adst
    e = asrc + adst
    e = jnp.where(e >= 0.0, e, 0.2 * e)
    sw_ref[...] = jnp.exp(e)


@functools.cache
def _get_tc_dense():
  return pl.pallas_call(
    _tc_dense_body,
    out_shape=(jax.ShapeDtypeStruct((N, D), jnp.float32),
               jax.ShapeDtypeStruct((N, 1), jnp.float32),
               jax.ShapeDtypeStruct((N, 1), jnp.float32),
               jax.ShapeDtypeStruct((N, 1), jnp.float32)),
  )


def _tc_combine_body(slope, acc0_ref, acc1_ref, den0_ref, den1_ref,
                     sw_ref, h_ref, b_ref, out_ref):
    den = den0_ref[...] + den1_ref[...] + sw_ref[...] + 1e-16
    num = acc0_ref[...] + acc1_ref[...] + sw_ref[...] * h_ref[...]
    out = num / den + b_ref[...]
    if slope is not None:
        out = jnp.where(out >= 0.0, out, slope * out)
    out_ref[...] = out


@functools.cache
def _make_combine(slope):
    return pl.pallas_call(
        functools.partial(_tc_combine_body, slope),
        out_shape=jax.ShapeDtypeStruct((N, D), jnp.float32),
    )


def _gat_layer(x, src2d, dst2d, W, a_s, a_d, b, slope):
    h, asrc, adst, sw = _get_tc_dense()(x, W, a_s.reshape(1, D), a_d.reshape(1, D))
    acc, den = _get_sc_edge()(h, src2d, dst2d, asrc.reshape(N), adst.reshape(N))
    return _make_combine(slope)(acc[0], acc[1], den[0, :N].reshape(N, 1),
                                den[1, :N].reshape(N, 1), sw, h, b.reshape(1, D))


def kernel(x, edge_index, W1, a_src1, a_dst1, b1, W2, a_src2, a_dst2, b2):
    src2d = jnp.concatenate([edge_index[0], edge_index[0][:EPAD - E]])
    src2d = src2d.reshape(EPAD // CH, CH)
    dst2d = jnp.concatenate([edge_index[1], edge_index[1][:EPAD - E]])
    dst2d = dst2d.reshape(EPAD // CH, CH)
    h1 = _gat_layer(x, src2d, dst2d, W1, a_src1, a_dst1, b1, 0.01)
    out = _gat_layer(h1, src2d, dst2d, W2, a_src2, a_dst2, b2, None)
    return out
